# Initial kernel scaffold; baseline (speedup 1.0000x reference)
import jax
import jax.numpy as jnp
from jax.experimental import pallas as pl


def kernel(data, segment_ids):
    raise NotImplementedError("write your pallas kernel here")


# speedup vs baseline: 3.3436x; 3.3436x over previous
import functools

import jax
import jax.numpy as jnp
from jax import lax
from jax.experimental import pallas as pl
from jax.experimental.pallas import tpu as pltpu
from jax.experimental.pallas import tpu_sc as plsc

N_POINTS = 320000
N_SEG = 10000
D = 128
L = 16
NC = 2
NS = 16
NW = NC * NS
P_PER_TILE = N_POINTS // NW
CHUNK = 80
N_CHUNKS = P_PER_TILE // CHUNK
SEG_PER_SUB = 632


def _accumulate(data, ids, zacc, acc_out, cnt_out,
                rows_v, idx_v, ones_v, acc_sh):
    c = lax.axis_index("c")
    s = lax.axis_index("s")
    t = c * NS + s
    base = t * P_PER_TILE
    zs = jnp.minimum(s * SEG_PER_SUB, N_SEG - SEG_PER_SUB)

    pltpu.sync_copy(zacc.at[pl.ds(zs, SEG_PER_SUB)],
                    acc_sh.at[pl.ds(zs, SEG_PER_SUB)])

    def fill_ones(i, _):
        for k in range(D // L):
            ones_v[i, pl.ds(k * L, L)] = jnp.ones((L,), jnp.float32)
        return 0
    lax.fori_loop(0, CHUNK, fill_ones, 0)

    plsc.subcore_barrier()

    def body_a(j, _):
        off = base + j * CHUNK
        pltpu.sync_copy(ids.at[pl.ds(off, CHUNK)], idx_v)
        pltpu.sync_copy(data.at[pl.ds(off, CHUNK)], rows_v)
        pltpu.sync_copy(rows_v, acc_sh.at[idx_v], add=True)
        return 0
    lax.fori_loop(0, N_CHUNKS, body_a, 0)

    plsc.subcore_barrier()
    pltpu.sync_copy(acc_sh.at[pl.ds(zs, SEG_PER_SUB)],
                    acc_out.at[pl.ds(c * N_SEG + zs, SEG_PER_SUB)])
    plsc.subcore_barrier()

    pltpu.sync_copy(zacc.at[pl.ds(zs, SEG_PER_SUB)],
                    acc_sh.at[pl.ds(zs, SEG_PER_SUB)])
    plsc.subcore_barrier()

    def body_b(j, _):
        off = base + j * CHUNK
        pltpu.sync_copy(ids.at[pl.ds(off, CHUNK)], idx_v)
        pltpu.sync_copy(ones_v, acc_sh.at[idx_v], add=True)
        return 0
    lax.fori_loop(0, N_CHUNKS, body_b, 0)

    plsc.subcore_barrier()
    pltpu.sync_copy(acc_sh.at[pl.ds(zs, SEG_PER_SUB)],
                    cnt_out.at[pl.ds(c * N_SEG + zs, SEG_PER_SUB)])


def _divide(a0, a1, c0, c1, o):
    cnt = jnp.maximum(c0[...] + c1[...], jnp.float32(1.0))
    o[...] = (a0[...] + a1[...]) / cnt


def kernel(data, segment_ids):
    mesh = plsc.VectorSubcoreMesh(core_axis_name="c", subcore_axis_name="s",
                                  num_cores=NC, num_subcores=NS)
    ids = segment_ids.astype(jnp.int32)
    zacc = jnp.zeros((N_SEG, D), jnp.float32)

    acc_pair, cnt_pair = pl.kernel(
        _accumulate,
        out_type=(
            jax.ShapeDtypeStruct((NC * N_SEG, D), jnp.float32),
            jax.ShapeDtypeStruct((NC * N_SEG, D), jnp.float32),
        ),
        mesh=mesh,
        scratch_types=[
            pltpu.VMEM((CHUNK, D), jnp.float32),
            pltpu.VMEM((CHUNK,), jnp.int32),
            pltpu.VMEM((CHUNK, D), jnp.float32),
            pltpu.VMEM_SHARED((N_SEG, D), jnp.float32),
        ],
    )(data, ids, zacc)

    blk = 1000
    grid = N_SEG // blk
    spec = pl.BlockSpec((blk, D), lambda i: (i, 0))
    out = pl.pallas_call(
        _divide,
        grid=(grid,),
        in_specs=[spec, spec, spec, spec],
        out_specs=spec,
        out_shape=jax.ShapeDtypeStruct((N_SEG, D), jnp.float32),
    )(acc_pair[:N_SEG], acc_pair[N_SEG:], cnt_pair[:N_SEG], cnt_pair[N_SEG:])

    return out


# speedup vs baseline: 5.2197x; 1.5611x over previous
import functools

import jax
import jax.numpy as jnp
from jax import lax
from jax.experimental import pallas as pl
from jax.experimental.pallas import tpu as pltpu
from jax.experimental.pallas import tpu_sc as plsc

N_POINTS = 320000
N_SEG = 10000
D = 128
L = 16
NC = 2
NS = 16
NW = NC * NS
P_PER_TILE = N_POINTS // NW
CHUNK = 80
N_CHUNKS = P_PER_TILE // CHUNK
SEG_PER_SUB = 632


def _accumulate(data, ids, zacc, acc_out, cnt_out,
                rows_v, idx2_v, ones_v, acc_sh, sem):
    c = lax.axis_index("c")
    s = lax.axis_index("s")
    t = c * NS + s
    base = t * P_PER_TILE
    zs = jnp.minimum(s * SEG_PER_SUB, N_SEG - SEG_PER_SUB)

    pltpu.sync_copy(zacc.at[pl.ds(zs, SEG_PER_SUB)],
                    acc_sh.at[pl.ds(zs, SEG_PER_SUB)])

    def idx_start(j, _):
        pltpu.async_copy(ids.at[pl.ds(base + j * CHUNK, CHUNK)],
                         idx2_v.at[j], sem)
        return 0
    lax.fori_loop(0, N_CHUNKS, idx_start, 0)

    def fill_ones(i, _):
        for k in range(D // L):
            ones_v[i, pl.ds(k * L, L)] = jnp.ones((L,), jnp.float32)
        return 0
    lax.fori_loop(0, CHUNK, fill_ones, 0)

    def idx_drain(j, _):
        pltpu.make_async_copy(ids.at[pl.ds(base, CHUNK)],
                              idx2_v.at[j], sem).wait()
        return 0
    lax.fori_loop(0, N_CHUNKS, idx_drain, 0)

    plsc.subcore_barrier()

    pltpu.async_copy(data.at[pl.ds(base, CHUNK)], rows_v.at[0], sem)

    def body_a(j, _):
        jm = lax.rem(j, 2)
        pltpu.make_async_copy(data.at[pl.ds(base, CHUNK)],
                              rows_v.at[jm], sem).wait()

        @pl.when(j + 1 < N_CHUNKS)
        def _():
            pltpu.async_copy(data.at[pl.ds(base + (j + 1) * CHUNK, CHUNK)],
                             rows_v.at[lax.rem(j + 1, 2)], sem)

        pltpu.sync_copy(rows_v.at[jm], acc_sh.at[idx2_v.at[j]], add=True)
        return 0
    lax.fori_loop(0, N_CHUNKS, body_a, 0)

    plsc.subcore_barrier()
    pltpu.sync_copy(acc_sh.at[pl.ds(zs, SEG_PER_SUB)],
                    acc_out.at[pl.ds(c * N_SEG + zs, SEG_PER_SUB)])
    plsc.subcore_barrier()

    pltpu.sync_copy(zacc.at[pl.ds(zs, SEG_PER_SUB)],
                    acc_sh.at[pl.ds(zs, SEG_PER_SUB)])
    plsc.subcore_barrier()

    def body_b(j, _):
        pltpu.sync_copy(ones_v, acc_sh.at[idx2_v.at[j]], add=True)
        return 0
    lax.fori_loop(0, N_CHUNKS, body_b, 0)

    plsc.subcore_barrier()
    pltpu.sync_copy(acc_sh.at[pl.ds(zs, SEG_PER_SUB)],
                    cnt_out.at[pl.ds(c * N_SEG + zs, SEG_PER_SUB)])


def _divide(a0, a1, c0, c1, o):
    cnt = jnp.maximum(c0[...] + c1[...], jnp.float32(1.0))
    o[...] = (a0[...] + a1[...]) / cnt


def kernel(data, segment_ids):
    mesh = plsc.VectorSubcoreMesh(core_axis_name="c", subcore_axis_name="s",
                                  num_cores=NC, num_subcores=NS)
    ids = segment_ids.astype(jnp.int32)
    zacc = jnp.zeros((N_SEG, D), jnp.float32)

    acc_pair, cnt_pair = pl.kernel(
        _accumulate,
        out_type=(
            jax.ShapeDtypeStruct((NC * N_SEG, D), jnp.float32),
            jax.ShapeDtypeStruct((NC * N_SEG, D), jnp.float32),
        ),
        mesh=mesh,
        scratch_types=[
            pltpu.VMEM((2, CHUNK, D), jnp.float32),
            pltpu.VMEM((N_CHUNKS, CHUNK), jnp.int32),
            pltpu.VMEM((CHUNK, D), jnp.float32),
            pltpu.VMEM_SHARED((N_SEG, D), jnp.float32),
            pltpu.SemaphoreType.DMA,
        ],
    )(data, ids, zacc)

    blk = 1000
    grid = N_SEG // blk
    spec = pl.BlockSpec((blk, D), lambda i: (i, 0))
    out = pl.pallas_call(
        _divide,
        grid=(grid,),
        in_specs=[spec, spec, spec, spec],
        out_specs=spec,
        out_shape=jax.ShapeDtypeStruct((N_SEG, D), jnp.float32),
    )(acc_pair[:N_SEG], acc_pair[N_SEG:], cnt_pair[:N_SEG], cnt_pair[N_SEG:])

    return out


# speedup vs baseline: 5.2656x; 1.0088x over previous
import functools

import jax
import jax.numpy as jnp
from jax import lax
from jax.experimental import pallas as pl
from jax.experimental.pallas import tpu as pltpu
from jax.experimental.pallas import tpu_sc as plsc

N_POINTS = 320000
N_SEG = 10000
D = 128
L = 16
NC = 2
NS = 16
NW = NC * NS
P_PER_TILE = N_POINTS // NW
CHUNK = 80
N_CHUNKS = P_PER_TILE // CHUNK
SEG_PER_SUB = 632


def _accumulate(data, ids, zacc, acc_out, cnt_out,
                rows_v, idx2_v, ones_v, acc_sh, sem):
    c = lax.axis_index("c")
    s = lax.axis_index("s")
    t = c * NS + s
    base = t * P_PER_TILE
    zs = jnp.minimum(s * SEG_PER_SUB, N_SEG - SEG_PER_SUB)

    pltpu.sync_copy(zacc.at[pl.ds(zs, SEG_PER_SUB)],
                    acc_sh.at[pl.ds(zs, SEG_PER_SUB)])

    def idx_start(j, _):
        pltpu.async_copy(ids.at[pl.ds(base + j * CHUNK, CHUNK)],
                         idx2_v.at[j], sem)
        return 0
    lax.fori_loop(0, N_CHUNKS, idx_start, 0)

    def fill_ones(i, _):
        for k in range(D // L):
            ones_v[i, pl.ds(k * L, L)] = jnp.ones((L,), jnp.float32)
        return 0
    lax.fori_loop(0, CHUNK, fill_ones, 0)

    def idx_drain(j, _):
        pltpu.make_async_copy(ids.at[pl.ds(base, CHUNK)],
                              idx2_v.at[j], sem).wait()
        return 0
    lax.fori_loop(0, N_CHUNKS, idx_drain, 0)

    plsc.subcore_barrier()

    pltpu.async_copy(data.at[pl.ds(base, CHUNK)], rows_v.at[0], sem)

    def body_a(j, _):
        jm = lax.rem(j, 2)
        pltpu.make_async_copy(data.at[pl.ds(base, CHUNK)],
                              rows_v.at[jm], sem).wait()

        @pl.when(j + 1 < N_CHUNKS)
        def _():
            pltpu.async_copy(data.at[pl.ds(base + (j + 1) * CHUNK, CHUNK)],
                             rows_v.at[lax.rem(j + 1, 2)], sem)

        pltpu.sync_copy(rows_v.at[jm], acc_sh.at[idx2_v.at[j]], add=True)
        return 0
    lax.fori_loop(0, N_CHUNKS, body_a, 0)

    plsc.subcore_barrier()
    pltpu.sync_copy(acc_sh.at[pl.ds(zs, SEG_PER_SUB)],
                    acc_out.at[pl.ds(c * N_SEG + zs, SEG_PER_SUB)])
    plsc.subcore_barrier()

    pltpu.sync_copy(zacc.at[pl.ds(zs, SEG_PER_SUB)],
                    acc_sh.at[pl.ds(zs, SEG_PER_SUB)])
    plsc.subcore_barrier()

    WIN = 24

    def body_b(j, _):
        pltpu.make_async_copy(ones_v, acc_sh.at[idx2_v.at[j]],
                              sem).start(add=True)

        @pl.when(j >= WIN)
        def _():
            pltpu.make_async_copy(ones_v, acc_sh.at[idx2_v.at[j]], sem).wait()
        return 0
    lax.fori_loop(0, N_CHUNKS, body_b, 0)

    def drain_b(j, _):
        pltpu.make_async_copy(ones_v, acc_sh.at[idx2_v.at[j]], sem).wait()
        return 0
    lax.fori_loop(0, WIN, drain_b, 0)

    plsc.subcore_barrier()
    pltpu.sync_copy(acc_sh.at[pl.ds(zs, SEG_PER_SUB)],
                    cnt_out.at[pl.ds(c * N_SEG + zs, SEG_PER_SUB)])


def _divide(a0, a1, c0, c1, o):
    cnt = jnp.maximum(c0[...] + c1[...], jnp.float32(1.0))
    o[...] = (a0[...] + a1[...]) / cnt


def kernel(data, segment_ids):
    mesh = plsc.VectorSubcoreMesh(core_axis_name="c", subcore_axis_name="s",
                                  num_cores=NC, num_subcores=NS)
    ids = segment_ids.astype(jnp.int32)
    zacc = jnp.zeros((N_SEG, D), jnp.float32)

    acc_pair, cnt_pair = pl.kernel(
        _accumulate,
        out_type=(
            jax.ShapeDtypeStruct((NC * N_SEG, D), jnp.float32),
            jax.ShapeDtypeStruct((NC * N_SEG, D), jnp.float32),
        ),
        mesh=mesh,
        scratch_types=[
            pltpu.VMEM((2, CHUNK, D), jnp.float32),
            pltpu.VMEM((N_CHUNKS, CHUNK), jnp.int32),
            pltpu.VMEM((CHUNK, D), jnp.float32),
            pltpu.VMEM_SHARED((N_SEG, D), jnp.float32),
            pltpu.SemaphoreType.DMA,
        ],
    )(data, ids, zacc)

    blk = 1000
    grid = N_SEG // blk
    spec = pl.BlockSpec((blk, D), lambda i: (i, 0))
    out = pl.pallas_call(
        _divide,
        grid=(grid,),
        in_specs=[spec, spec, spec, spec],
        out_specs=spec,
        out_shape=jax.ShapeDtypeStruct((N_SEG, D), jnp.float32),
    )(acc_pair[:N_SEG], acc_pair[N_SEG:], cnt_pair[:N_SEG], cnt_pair[N_SEG:])

    return out


# speedup vs baseline: 7.4016x; 1.4057x over previous
import functools

import jax
import jax.numpy as jnp
from jax import lax
from jax.experimental import pallas as pl
from jax.experimental.pallas import tpu as pltpu
from jax.experimental.pallas import tpu_sc as plsc

N_POINTS = 320000
N_SEG = 10000
D = 128
L = 16
NC = 2
NS = 16
NW = NC * NS
P_PER_TILE = N_POINTS // NW
CHUNK = 80
N_CHUNKS = P_PER_TILE // CHUNK
VPC = CHUNK // L
SEG_PER_SUB = 632


def _accumulate(data, ids, zacc, acc_out, cnt_out,
                rows_v, idx2_v, idxf_v, cnt_v, mx_v, acc_sh, sem, sem2, sem3):
    c = lax.axis_index("c")
    s = lax.axis_index("s")
    t = c * NS + s
    base = t * P_PER_TILE
    zs = jnp.minimum(s * SEG_PER_SUB, N_SEG - SEG_PER_SUB)

    pltpu.sync_copy(zacc.at[pl.ds(zs, SEG_PER_SUB)],
                    acc_sh.at[pl.ds(zs, SEG_PER_SUB)])

    pltpu.async_copy(ids.at[pl.ds(base, P_PER_TILE)],
                     idxf_v.at[pl.ds(0, P_PER_TILE)], sem)
    pltpu.async_copy(ids.at[pl.ds(base, CHUNK)], idx2_v.at[0], sem2)
    pltpu.async_copy(ids.at[pl.ds(base + CHUNK, CHUNK)], idx2_v.at[1], sem2)

    def zero_cnt(i, _):
        cnt_v[pl.ds(i * L, L)] = jnp.zeros((L,), jnp.float32)
        return 0
    lax.fori_loop(0, N_SEG // L, zero_cnt, 0)
    mx_v[pl.ds(0, L)] = jnp.full((L,), -1, jnp.int32)

    pltpu.make_async_copy(ids.at[pl.ds(base, P_PER_TILE)],
                          idxf_v.at[pl.ds(0, P_PER_TILE)], sem).wait()

    idxf_v[pl.ds(P_PER_TILE, L)] = jnp.full((L,), -1, jnp.int32)

    plsc.subcore_barrier()

    pltpu.async_copy(data.at[pl.ds(base, CHUNK)], rows_v.at[0], sem)
    iota = lax.iota(jnp.int32, L)

    def runlen_vec(m, prev_end):
        pos0 = m * L
        q = idxf_v[pl.ds(pos0, L)]
        nxt = idxf_v[pl.ds(pos0 + 1, L)]
        b_end = q != nxt
        pos = iota + pos0
        ends = jnp.where(b_end, pos, -1)
        mx_v[pl.ds(1, L)] = plsc.cummax(ends)
        excl = mx_v[pl.ds(0, L)]
        prev = jnp.maximum(excl, jnp.full((L,), prev_end))
        counts = (pos - prev).astype(jnp.float32)
        plsc.store_scatter(cnt_v, [q], counts, mask=b_end)
        return jnp.max(jnp.where(b_end, pos, jnp.full((L,), prev_end)))

    def body_a(j, prev_end):
        jm = lax.rem(j, 2)
        jr = lax.rem(j, 4)
        pltpu.make_async_copy(data.at[pl.ds(base, CHUNK)],
                              rows_v.at[jm], sem).wait()
        pltpu.make_async_copy(ids.at[pl.ds(base, CHUNK)],
                              idx2_v.at[jr], sem2).wait()

        @pl.when(j + 1 < N_CHUNKS)
        def _():
            pltpu.async_copy(data.at[pl.ds(base + (j + 1) * CHUNK, CHUNK)],
                             rows_v.at[lax.rem(j + 1, 2)], sem)

        @pl.when(j + 2 < N_CHUNKS)
        def _():
            pltpu.async_copy(ids.at[pl.ds(base + (j + 2) * CHUNK, CHUNK)],
                             idx2_v.at[lax.rem(j + 2, 4)], sem2)

        pltpu.make_async_copy(rows_v.at[jm], acc_sh.at[idx2_v.at[jr]],
                              sem3).start(add=True)
        for v in range(VPC):
            prev_end = runlen_vec(j * VPC + v, prev_end)
        pltpu.make_async_copy(rows_v.at[jm], acc_sh.at[idx2_v.at[jr]],
                              sem3).wait()
        return prev_end

    lax.fori_loop(0, N_CHUNKS, body_a, jnp.int32(-1))

    plsc.subcore_barrier()
    pltpu.sync_copy(acc_sh.at[pl.ds(zs, SEG_PER_SUB)],
                    acc_out.at[pl.ds(c * N_SEG + zs, SEG_PER_SUB)])
    pltpu.sync_copy(cnt_v, cnt_out.at[pl.ds(t * N_SEG, N_SEG)])


def _divide(a0, a1, cnt, o):
    csum = jnp.maximum(jnp.sum(cnt[...], axis=0), jnp.float32(1.0))
    o[...] = (a0[...] + a1[...]) / csum[:, None]


def kernel(data, segment_ids):
    mesh = plsc.VectorSubcoreMesh(core_axis_name="c", subcore_axis_name="s",
                                  num_cores=NC, num_subcores=NS)
    ids = segment_ids.astype(jnp.int32)
    zacc = jnp.zeros((N_SEG, D), jnp.float32)

    acc_pair, cnt_flat = pl.kernel(
        _accumulate,
        out_type=(
            jax.ShapeDtypeStruct((NC * N_SEG, D), jnp.float32),
            jax.ShapeDtypeStruct((NW * N_SEG,), jnp.float32),
        ),
        mesh=mesh,
        scratch_types=[
            pltpu.VMEM((2, CHUNK, D), jnp.float32),
            pltpu.VMEM((4, CHUNK), jnp.int32),
            pltpu.VMEM((P_PER_TILE + L,), jnp.int32),
            pltpu.VMEM((N_SEG,), jnp.float32),
            pltpu.VMEM((L + 8,), jnp.int32),
            pltpu.VMEM_SHARED((N_SEG, D), jnp.float32),
            pltpu.SemaphoreType.DMA,
            pltpu.SemaphoreType.DMA,
            pltpu.SemaphoreType.DMA,
        ],
        compiler_params=pltpu.CompilerParams(needs_layout_passes=False),
    )(data, ids, zacc)

    cnts = cnt_flat.reshape(NW, N_SEG)

    out = pl.pallas_call(
        _divide,
        out_shape=jax.ShapeDtypeStruct((N_SEG, D), jnp.float32),
    )(acc_pair[:N_SEG], acc_pair[N_SEG:], cnts)

    return out
